# baseline (speedup 1.0000x reference)
import functools

import jax
import jax.numpy as jnp
from jax import lax
from jax.experimental import pallas as pl
from jax.experimental.pallas import tpu as pltpu
from jax.experimental.pallas import tpu_sc as plsc

N = 10000
D = 128
G = 64
NCLS = 10

NC = 2
NS = 16
NW = NC * NS
CH = 128
K = 3

TILE_ROWS = 640
S_ROWS = NS * TILE_ROWS


def _agg_kernel(tpw, h_hbm, gs_hbm, sd_hbm, out0, out1,
                sidx0, sidx1, didx0, didx1, rows0, rows1, zbuf, aggtab,
                sg0, sg1, ss0, ss1, sd0, sd1):
    cid = lax.axis_index("c")
    sid = lax.axis_index("s")
    wid = cid * NS + sid
    base = wid * tpw

    sidx = (sidx0, sidx1)
    didx = (didx0, didx1)
    rows = (rows0, rows1)
    sem_g = (sg0, sg1)
    sem_s = (ss0, ss1)
    sem_d = (sd0, sd1)

    def idx_start(i, b):
        off = pl.multiple_of((base + i) * CH, CH)
        pltpu.async_copy(gs_hbm.at[pl.ds(off, CH)], sidx[b], sem_s[b])
        pltpu.async_copy(sd_hbm.at[pl.ds(off, CH)], didx[b], sem_d[b])

    def idx_wait(b):
        pltpu.make_async_copy(gs_hbm.at[pl.ds(0, CH)], sidx[b],
                              sem_s[b]).wait()

    def didx_wait(b):
        pltpu.make_async_copy(sd_hbm.at[pl.ds(0, CH)], didx[b],
                              sem_d[b]).wait()

    def gather_wait(b):
        pltpu.make_async_copy(h_hbm.at[sidx[b]], rows[b], sem_g[b]).wait()

    zero = jnp.zeros((16,), jnp.float32)
    for r in range(16):
        for k in range(D // 16):
            zbuf[r, pl.ds(k * 16, 16)] = zero

    idx_start(0, 0)
    idx_start(1, 1)

    for j in range(TILE_ROWS // 16):
        pltpu.sync_copy(zbuf, aggtab.at[pl.ds(sid * TILE_ROWS + j * 16, 16)])
    plsc.subcore_barrier()

    idx_wait(0)
    pltpu.async_copy(h_hbm.at[sidx[0]], rows[0], sem_g[0])

    def pair_body(g, carry):
        for b in range(2):
            i = 2 * g + b

            @pl.when(i + 1 < tpw)
            def _():
                idx_wait(1 - b)
                pltpu.async_copy(h_hbm.at[sidx[1 - b]], rows[1 - b],
                                 sem_g[1 - b])

            gather_wait(b)
            didx_wait(b)
            pltpu.sync_copy(rows[b], aggtab.at[didx[b]], add=True)

            @pl.when(i + 2 < tpw)
            def _():
                idx_start(i + 2, b)
        return carry

    lax.fori_loop(0, tpw // 2, pair_body, 0)
    plsc.subcore_barrier()

    row0 = sid * TILE_ROWS

    @pl.when(cid == 0)
    def _():
        pltpu.sync_copy(aggtab.at[pl.ds(row0, TILE_ROWS)],
                        out0.at[pl.ds(row0, TILE_ROWS)])

    @pl.when(cid == 1)
    def _():
        pltpu.sync_copy(aggtab.at[pl.ds(row0, TILE_ROWS)],
                        out1.at[pl.ds(row0, TILE_ROWS)])


def _make_agg(tpw):
    mesh = plsc.VectorSubcoreMesh(core_axis_name="c", subcore_axis_name="s")
    return pl.kernel(
        functools.partial(_agg_kernel, tpw),
        out_type=(jax.ShapeDtypeStruct((S_ROWS, D), jnp.float32),
                  jax.ShapeDtypeStruct((S_ROWS, D), jnp.float32)),
        mesh=mesh,
        scratch_types=[
            pltpu.VMEM((CH,), jnp.int32),
            pltpu.VMEM((CH,), jnp.int32),
            pltpu.VMEM((CH,), jnp.int32),
            pltpu.VMEM((CH,), jnp.int32),
            pltpu.VMEM((CH, D), jnp.float32),
            pltpu.VMEM((CH, D), jnp.float32),
            pltpu.VMEM((16, D), jnp.float32),
            pltpu.VMEM_SHARED((S_ROWS, D), jnp.float32),
            pltpu.SemaphoreType.DMA,
            pltpu.SemaphoreType.DMA,
            pltpu.SemaphoreType.DMA,
            pltpu.SemaphoreType.DMA,
            pltpu.SemaphoreType.DMA,
            pltpu.SemaphoreType.DMA,
        ],
    )


ROWS_BLK = 1000
N_BLKS = N // ROWS_BLK


def _mlp_block(h_ref, p0_ref, p1_ref, wa_ref, ba_ref, wb_ref, bb_ref, out_ref):
    h2 = h_ref[...] + p0_ref[...] + p1_ref[...]
    z = jnp.maximum(
        jnp.dot(h2, wa_ref[...], preferred_element_type=jnp.float32)
        + ba_ref[...], 0.0)
    out_ref[...] = (jnp.dot(z, wb_ref[...], preferred_element_type=jnp.float32)
                    + bb_ref[...])


def _mlp1(h, p0, p1, wa, ba, wb, bb):
    blk = lambda i: (i, 0)
    fixed = lambda i: (0, 0)
    return pl.pallas_call(
        _mlp_block,
        grid=(N_BLKS,),
        in_specs=[
            pl.BlockSpec((ROWS_BLK, D), blk),
            pl.BlockSpec((ROWS_BLK, D), blk),
            pl.BlockSpec((ROWS_BLK, D), blk),
            pl.BlockSpec((D, D), fixed),
            pl.BlockSpec((1, D), fixed),
            pl.BlockSpec((D, D), fixed),
            pl.BlockSpec((1, D), fixed),
        ],
        out_specs=pl.BlockSpec((ROWS_BLK, D), blk),
        out_shape=jax.ShapeDtypeStruct((N, D), jnp.float32),
    )(h, p0, p1, wa, ba, wb, bb)


def _mlp2_pool_block(h_ref, p0_ref, p1_ref, wa_ref, ba_ref, wb_ref, bb_ref,
                     batch_ref, pool_ref):
    @pl.when(pl.program_id(0) == 0)
    def _():
        pool_ref[...] = jnp.full((G, D), -jnp.inf, jnp.float32)

    h2 = h_ref[...] + p0_ref[...] + p1_ref[...]
    z = jnp.maximum(
        jnp.dot(h2, wa_ref[...], preferred_element_type=jnp.float32)
        + ba_ref[...], 0.0)
    out = (jnp.dot(z, wb_ref[...], preferred_element_type=jnp.float32)
           + bb_ref[...])
    bids = batch_ref[...]
    cur = pool_ref[...]
    upd = []
    for g in range(G):
        m = bids == g
        upd.append(jnp.max(jnp.where(m, out, -jnp.inf), axis=0))
    pool_ref[...] = jnp.maximum(cur, jnp.stack(upd, axis=0))


def _mlp2_pool(h, p0, p1, wa, ba, wb, bb, batch3):
    blk = lambda i: (i, 0)
    fixed = lambda i: (0, 0)
    return pl.pallas_call(
        _mlp2_pool_block,
        grid=(N_BLKS,),
        in_specs=[
            pl.BlockSpec((ROWS_BLK, D), blk),
            pl.BlockSpec((ROWS_BLK, D), blk),
            pl.BlockSpec((ROWS_BLK, D), blk),
            pl.BlockSpec((D, D), fixed),
            pl.BlockSpec((1, D), fixed),
            pl.BlockSpec((D, D), fixed),
            pl.BlockSpec((1, D), fixed),
            pl.BlockSpec((ROWS_BLK, 1), lambda i: (i, 0)),
        ],
        out_specs=pl.BlockSpec((G, D), fixed),
        out_shape=jax.ShapeDtypeStruct((G, D), jnp.float32),
    )(h, p0, p1, wa, ba, wb, bb, batch3)


def _final_block(pool_ref, wf1_ref, bf1_ref, wf2_ref, bf2_ref, out_ref):
    pooled = pool_ref[...]
    pooled = jnp.where(jnp.isfinite(pooled), pooled, 0.0)
    z = jnp.maximum(
        jnp.dot(pooled, wf1_ref[...], preferred_element_type=jnp.float32)
        + bf1_ref[...], 0.0)
    o = (jnp.dot(z, wf2_ref[...], preferred_element_type=jnp.float32)
         + bf2_ref[...])
    m = jnp.max(o, axis=1, keepdims=True)
    s = o - m
    lse = jnp.log(jnp.sum(jnp.exp(s), axis=1, keepdims=True))
    out_ref[...] = s - lse


def _final(pooled, wf1, bf1, wf2, bf2):
    return pl.pallas_call(
        _final_block,
        out_shape=jax.ShapeDtypeStruct((G, NCLS), jnp.float32),
    )(pooled, wf1, bf1, wf2, bf2)


def kernel(x, edge_index, batch, W1a, b1a, W1b, b1b, W2a, b2a, W2b, b2b,
           Wf1, bf1, Wf2, bf2):
    src = edge_index[0]
    dst = edge_index[1]
    s2 = jnp.concatenate([src, dst])
    d2 = jnp.concatenate([dst, src])
    key = jnp.sort(d2 * N + s2)
    dup = jnp.concatenate([jnp.zeros((1,), bool), key[1:] == key[:-1]])
    gs = (key % N).astype(jnp.int32)
    sd = jnp.where(dup, N, key // N).astype(jnp.int32)
    key2 = jnp.sort(s2 * N + d2)
    gs = gs + (key2[0:1] // jnp.int32(2 ** 30)).astype(jnp.int32)

    e2 = gs.shape[0]
    tpw = -(-e2 // (NW * CH))
    tpw = tpw + (tpw % 2)
    e2p = NW * tpw * CH
    pad = e2p - e2
    gs = jnp.concatenate([gs, jnp.zeros((pad,), jnp.int32)])
    sd = jnp.concatenate([sd, jnp.full((pad,), N, jnp.int32)])

    agg = _make_agg(tpw)

    ba1 = b1a.reshape(1, D)
    bb1 = b1b.reshape(1, D)
    ba2 = b2a.reshape(1, D)
    bb2 = b2b.reshape(1, D)
    batch3 = batch.astype(jnp.int32).reshape(N, 1)

    p0, p1 = agg(x, gs, sd)
    h = _mlp1(x, p0, p1, W1a, ba1, W1b, bb1)
    q0, q1 = agg(h, gs, sd)
    pooled = _mlp2_pool(h, q0, q1, W2a, ba2, W2b, bb2, batch3)
    out = _final(pooled, Wf1, bf1.reshape(1, 64), Wf2, bf2.reshape(1, NCLS))
    return out


# speedup vs baseline: 1.8535x; 1.8535x over previous
import functools

import jax
import jax.numpy as jnp
from jax import lax
from jax.experimental import pallas as pl
from jax.experimental.pallas import tpu as pltpu
from jax.experimental.pallas import tpu_sc as plsc

N = 10000
D = 128
G = 64
NCLS = 10

NC = 2
NS = 16
NW = NC * NS
CH = 128
K = 3

TILE_ROWS = 640
S_ROWS = NS * TILE_ROWS


def _agg_kernel(tpw, h_hbm, gs_hbm, sd_hbm, out0, out1,
                sidx0, sidx1, didx0, didx1, rows0, rows1, zbuf, aggtab,
                sg0, sg1, ss0, ss1, sd0, sd1):
    cid = lax.axis_index("c")
    sid = lax.axis_index("s")
    wid = cid * NS + sid
    base = wid * tpw

    sidx = (sidx0, sidx1)
    didx = (didx0, didx1)
    rows = (rows0, rows1)
    sem_g = (sg0, sg1)
    sem_s = (ss0, ss1)
    sem_d = (sd0, sd1)

    def idx_start(i, b):
        off = pl.multiple_of((base + i) * CH, CH)
        pltpu.async_copy(gs_hbm.at[pl.ds(off, CH)], sidx[b], sem_s[b])
        pltpu.async_copy(sd_hbm.at[pl.ds(off, CH)], didx[b], sem_d[b])

    def idx_wait(b):
        pltpu.make_async_copy(gs_hbm.at[pl.ds(0, CH)], sidx[b],
                              sem_s[b]).wait()

    def didx_wait(b):
        pltpu.make_async_copy(sd_hbm.at[pl.ds(0, CH)], didx[b],
                              sem_d[b]).wait()

    def gather_wait(b):
        pltpu.make_async_copy(h_hbm.at[sidx[b]], rows[b], sem_g[b]).wait()

    zero = jnp.zeros((16,), jnp.float32)
    for r in range(16):
        for k in range(D // 16):
            zbuf[r, pl.ds(k * 16, 16)] = zero

    idx_start(0, 0)
    idx_start(1, 1)

    for j in range(TILE_ROWS // 16):
        pltpu.sync_copy(zbuf, aggtab.at[pl.ds(sid * TILE_ROWS + j * 16, 16)])
    plsc.subcore_barrier()

    idx_wait(0)
    pltpu.async_copy(h_hbm.at[sidx[0]], rows[0], sem_g[0])

    def pair_body(g, carry):
        for b in range(2):
            i = 2 * g + b

            @pl.when(i + 1 < tpw)
            def _():
                idx_wait(1 - b)
                pltpu.async_copy(h_hbm.at[sidx[1 - b]], rows[1 - b],
                                 sem_g[1 - b])

            gather_wait(b)
            didx_wait(b)
            pltpu.sync_copy(rows[b], aggtab.at[didx[b]], add=True)

            @pl.when(i + 2 < tpw)
            def _():
                idx_start(i + 2, b)
        return carry

    lax.fori_loop(0, tpw // 2, pair_body, 0)
    plsc.subcore_barrier()

    row0 = sid * TILE_ROWS

    @pl.when(cid == 0)
    def _():
        pltpu.sync_copy(aggtab.at[pl.ds(row0, TILE_ROWS)],
                        out0.at[pl.ds(row0, TILE_ROWS)])

    @pl.when(cid == 1)
    def _():
        pltpu.sync_copy(aggtab.at[pl.ds(row0, TILE_ROWS)],
                        out1.at[pl.ds(row0, TILE_ROWS)])


def _make_agg(tpw):
    mesh = plsc.VectorSubcoreMesh(core_axis_name="c", subcore_axis_name="s")
    return pl.kernel(
        functools.partial(_agg_kernel, tpw),
        out_type=(jax.ShapeDtypeStruct((S_ROWS, D), jnp.float32),
                  jax.ShapeDtypeStruct((S_ROWS, D), jnp.float32)),
        mesh=mesh,
        scratch_types=[
            pltpu.VMEM((CH,), jnp.int32),
            pltpu.VMEM((CH,), jnp.int32),
            pltpu.VMEM((CH,), jnp.int32),
            pltpu.VMEM((CH,), jnp.int32),
            pltpu.VMEM((CH, D), jnp.float32),
            pltpu.VMEM((CH, D), jnp.float32),
            pltpu.VMEM((16, D), jnp.float32),
            pltpu.VMEM_SHARED((S_ROWS, D), jnp.float32),
            pltpu.SemaphoreType.DMA,
            pltpu.SemaphoreType.DMA,
            pltpu.SemaphoreType.DMA,
            pltpu.SemaphoreType.DMA,
            pltpu.SemaphoreType.DMA,
            pltpu.SemaphoreType.DMA,
        ],
    )


ROWS_BLK = 1000
N_BLKS = N // ROWS_BLK


def _mlp_block(h_ref, p0_ref, p1_ref, wa_ref, ba_ref, wb_ref, bb_ref, out_ref):
    h2 = h_ref[...] + p0_ref[...] + p1_ref[...]
    z = jnp.maximum(
        jnp.dot(h2, wa_ref[...], preferred_element_type=jnp.float32)
        + ba_ref[...], 0.0)
    out_ref[...] = (jnp.dot(z, wb_ref[...], preferred_element_type=jnp.float32)
                    + bb_ref[...])


def _mlp1(h, p0, p1, wa, ba, wb, bb):
    blk = lambda i: (i, 0)
    fixed = lambda i: (0, 0)
    return pl.pallas_call(
        _mlp_block,
        grid=(N_BLKS,),
        in_specs=[
            pl.BlockSpec((ROWS_BLK, D), blk),
            pl.BlockSpec((ROWS_BLK, D), blk),
            pl.BlockSpec((ROWS_BLK, D), blk),
            pl.BlockSpec((D, D), fixed),
            pl.BlockSpec((1, D), fixed),
            pl.BlockSpec((D, D), fixed),
            pl.BlockSpec((1, D), fixed),
        ],
        out_specs=pl.BlockSpec((ROWS_BLK, D), blk),
        out_shape=jax.ShapeDtypeStruct((N, D), jnp.float32),
    )(h, p0, p1, wa, ba, wb, bb)


def _mlp2_pool_block(h_ref, p0_ref, p1_ref, wa_ref, ba_ref, wb_ref, bb_ref,
                     batch_ref, pool_ref):
    @pl.when(pl.program_id(0) == 0)
    def _():
        pool_ref[...] = jnp.full((G, D), -jnp.inf, jnp.float32)

    h2 = h_ref[...] + p0_ref[...] + p1_ref[...]
    z = jnp.maximum(
        jnp.dot(h2, wa_ref[...], preferred_element_type=jnp.float32)
        + ba_ref[...], 0.0)
    out = (jnp.dot(z, wb_ref[...], preferred_element_type=jnp.float32)
           + bb_ref[...])
    bids = batch_ref[...]
    cur = pool_ref[...]
    upd = []
    for g in range(G):
        m = bids == g
        upd.append(jnp.max(jnp.where(m, out, -jnp.inf), axis=0))
    pool_ref[...] = jnp.maximum(cur, jnp.stack(upd, axis=0))


def _mlp2_pool(h, p0, p1, wa, ba, wb, bb, batch3):
    blk = lambda i: (i, 0)
    fixed = lambda i: (0, 0)
    return pl.pallas_call(
        _mlp2_pool_block,
        grid=(N_BLKS,),
        in_specs=[
            pl.BlockSpec((ROWS_BLK, D), blk),
            pl.BlockSpec((ROWS_BLK, D), blk),
            pl.BlockSpec((ROWS_BLK, D), blk),
            pl.BlockSpec((D, D), fixed),
            pl.BlockSpec((1, D), fixed),
            pl.BlockSpec((D, D), fixed),
            pl.BlockSpec((1, D), fixed),
            pl.BlockSpec((ROWS_BLK, 1), lambda i: (i, 0)),
        ],
        out_specs=pl.BlockSpec((G, D), fixed),
        out_shape=jax.ShapeDtypeStruct((G, D), jnp.float32),
    )(h, p0, p1, wa, ba, wb, bb, batch3)


def _final_block(pool_ref, wf1_ref, bf1_ref, wf2_ref, bf2_ref, out_ref):
    pooled = pool_ref[...]
    pooled = jnp.where(jnp.isfinite(pooled), pooled, 0.0)
    z = jnp.maximum(
        jnp.dot(pooled, wf1_ref[...], preferred_element_type=jnp.float32)
        + bf1_ref[...], 0.0)
    o = (jnp.dot(z, wf2_ref[...], preferred_element_type=jnp.float32)
         + bf2_ref[...])
    m = jnp.max(o, axis=1, keepdims=True)
    s = o - m
    lse = jnp.log(jnp.sum(jnp.exp(s), axis=1, keepdims=True))
    out_ref[...] = s - lse


def _final(pooled, wf1, bf1, wf2, bf2):
    return pl.pallas_call(
        _final_block,
        out_shape=jax.ShapeDtypeStruct((G, NCLS), jnp.float32),
    )(pooled, wf1, bf1, wf2, bf2)


def kernel(x, edge_index, batch, W1a, b1a, W1b, b1b, W2a, b2a, W2b, b2b,
           Wf1, bf1, Wf2, bf2):
    src = edge_index[0]
    dst = edge_index[1]
    s2 = jnp.concatenate([src, dst])
    d2 = jnp.concatenate([dst, src])
    key = lax.sort(d2 * N + s2, is_stable=False)
    dup = jnp.concatenate([jnp.zeros((1,), bool), key[1:] == key[:-1]])
    gs = (key % N).astype(jnp.int32)
    sd = jnp.where(dup, N, key // N).astype(jnp.int32)

    e2 = gs.shape[0]
    tpw = -(-e2 // (NW * CH))
    tpw = tpw + (tpw % 2)
    e2p = NW * tpw * CH
    pad = e2p - e2
    gs = jnp.concatenate([gs, jnp.zeros((pad,), jnp.int32)])
    sd = jnp.concatenate([sd, jnp.full((pad,), N, jnp.int32)])

    agg = _make_agg(tpw)

    ba1 = b1a.reshape(1, D)
    bb1 = b1b.reshape(1, D)
    ba2 = b2a.reshape(1, D)
    bb2 = b2b.reshape(1, D)
    batch3 = batch.astype(jnp.int32).reshape(N, 1)

    p0, p1 = agg(x, gs, sd)
    h = _mlp1(x, p0, p1, W1a, ba1, W1b, bb1)
    q0, q1 = agg(h, gs, sd)
    pooled = _mlp2_pool(h, q0, q1, W2a, ba2, W2b, bb2, batch3)
    out = _final(pooled, Wf1, bf1.reshape(1, 64), Wf2, bf2.reshape(1, NCLS))
    return out


# speedup vs baseline: 1.8546x; 1.0006x over previous
import functools

import jax
import jax.numpy as jnp
from jax import lax
from jax.experimental import pallas as pl
from jax.experimental.pallas import tpu as pltpu
from jax.experimental.pallas import tpu_sc as plsc

N = 10000
D = 128
G = 64
NCLS = 10

NC = 2
NS = 16
NW = NC * NS
CH = 128

TILE_ROWS = 640
S_ROWS = NS * TILE_ROWS


def _agg_kernel(tpw, h_hbm, gs_hbm, sd_hbm, out0, out1,
                sidx0, sidx1, didx0, didx1, rows0, rows1, zbuf, aggtab,
                sg0, sg1, ss0, ss1, sd0, sd1):
    cid = lax.axis_index("c")
    sid = lax.axis_index("s")
    wid = cid * NS + sid
    base = wid * tpw

    sidx = (sidx0, sidx1)
    didx = (didx0, didx1)
    rows = (rows0, rows1)
    sem_g = (sg0, sg1)
    sem_s = (ss0, ss1)
    sem_d = (sd0, sd1)

    def idx_start(i, b):
        off = pl.multiple_of((base + i) * CH, CH)
        pltpu.async_copy(gs_hbm.at[pl.ds(off, CH)], sidx[b], sem_s[b])
        pltpu.async_copy(sd_hbm.at[pl.ds(off, CH)], didx[b], sem_d[b])

    def idx_wait(b):
        pltpu.make_async_copy(gs_hbm.at[pl.ds(0, CH)], sidx[b],
                              sem_s[b]).wait()

    def didx_wait(b):
        pltpu.make_async_copy(sd_hbm.at[pl.ds(0, CH)], didx[b],
                              sem_d[b]).wait()

    def gather_wait(b):
        pltpu.make_async_copy(h_hbm.at[sidx[b]], rows[b], sem_g[b]).wait()

    zero = jnp.zeros((16,), jnp.float32)
    for r in range(16):
        for k in range(D // 16):
            zbuf[r, pl.ds(k * 16, 16)] = zero

    idx_start(0, 0)
    idx_start(1, 1)

    for j in range(TILE_ROWS // 16):
        pltpu.sync_copy(zbuf, aggtab.at[pl.ds(sid * TILE_ROWS + j * 16, 16)])
    plsc.subcore_barrier()

    idx_wait(0)
    pltpu.async_copy(h_hbm.at[sidx[0]], rows[0], sem_g[0])

    def pair_body(g, carry):
        for b in range(2):
            i = 2 * g + b

            @pl.when(i + 1 < tpw)
            def _():
                idx_wait(1 - b)
                pltpu.async_copy(h_hbm.at[sidx[1 - b]], rows[1 - b],
                                 sem_g[1 - b])

            gather_wait(b)
            didx_wait(b)
            pltpu.sync_copy(rows[b], aggtab.at[didx[b]], add=True)

            @pl.when(i + 2 < tpw)
            def _():
                idx_start(i + 2, b)
        return carry

    lax.fori_loop(0, tpw // 2, pair_body, 0)
    plsc.subcore_barrier()

    row0 = sid * TILE_ROWS

    @pl.when(cid == 0)
    def _():
        pltpu.sync_copy(aggtab.at[pl.ds(row0, TILE_ROWS)],
                        out0.at[pl.ds(row0, TILE_ROWS)])

    @pl.when(cid == 1)
    def _():
        pltpu.sync_copy(aggtab.at[pl.ds(row0, TILE_ROWS)],
                        out1.at[pl.ds(row0, TILE_ROWS)])


def _make_agg(tpw):
    mesh = plsc.VectorSubcoreMesh(core_axis_name="c", subcore_axis_name="s")
    return pl.kernel(
        functools.partial(_agg_kernel, tpw),
        out_type=(jax.ShapeDtypeStruct((S_ROWS, D), jnp.float32),
                  jax.ShapeDtypeStruct((S_ROWS, D), jnp.float32)),
        mesh=mesh,
        scratch_types=[
            pltpu.VMEM((CH,), jnp.int32),
            pltpu.VMEM((CH,), jnp.int32),
            pltpu.VMEM((CH,), jnp.int32),
            pltpu.VMEM((CH,), jnp.int32),
            pltpu.VMEM((CH, D), jnp.float32),
            pltpu.VMEM((CH, D), jnp.float32),
            pltpu.VMEM((16, D), jnp.float32),
            pltpu.VMEM_SHARED((S_ROWS, D), jnp.float32),
            pltpu.SemaphoreType.DMA,
            pltpu.SemaphoreType.DMA,
            pltpu.SemaphoreType.DMA,
            pltpu.SemaphoreType.DMA,
            pltpu.SemaphoreType.DMA,
            pltpu.SemaphoreType.DMA,
        ],
    )


ROWS_BLK = 1000
N_BLKS = N // ROWS_BLK


def _mlp_block(h_ref, p0_ref, p1_ref, wa_ref, ba_ref, wb_ref, bb_ref, out_ref):
    h2 = h_ref[...] + p0_ref[...] + p1_ref[...]
    z = jnp.maximum(
        jnp.dot(h2, wa_ref[...], preferred_element_type=jnp.float32)
        + ba_ref[...], 0.0)
    out_ref[...] = (jnp.dot(z, wb_ref[...], preferred_element_type=jnp.float32)
                    + bb_ref[...])


def _mlp1(h, p0, p1, wa, ba, wb, bb):
    blk = lambda i: (i, 0)
    fixed = lambda i: (0, 0)
    return pl.pallas_call(
        _mlp_block,
        grid=(N_BLKS,),
        in_specs=[
            pl.BlockSpec((ROWS_BLK, D), blk),
            pl.BlockSpec((ROWS_BLK, D), blk),
            pl.BlockSpec((ROWS_BLK, D), blk),
            pl.BlockSpec((D, D), fixed),
            pl.BlockSpec((1, D), fixed),
            pl.BlockSpec((D, D), fixed),
            pl.BlockSpec((1, D), fixed),
        ],
        out_specs=pl.BlockSpec((ROWS_BLK, D), blk),
        out_shape=jax.ShapeDtypeStruct((N, D), jnp.float32),
    )(h, p0, p1, wa, ba, wb, bb)


def _mlp2_pool_block(h_ref, p0_ref, p1_ref, wa_ref, ba_ref, wb_ref, bb_ref,
                     batch_ref, pool_ref):
    @pl.when(pl.program_id(0) == 0)
    def _():
        pool_ref[...] = jnp.full((G, D), -jnp.inf, jnp.float32)

    h2 = h_ref[...] + p0_ref[...] + p1_ref[...]
    z = jnp.maximum(
        jnp.dot(h2, wa_ref[...], preferred_element_type=jnp.float32)
        + ba_ref[...], 0.0)
    out = (jnp.dot(z, wb_ref[...], preferred_element_type=jnp.float32)
           + bb_ref[...])
    bids = batch_ref[...]
    cur = pool_ref[...]
    upd = []
    for g in range(G):
        m = bids == g
        upd.append(jnp.max(jnp.where(m, out, -jnp.inf), axis=0))
    pool_ref[...] = jnp.maximum(cur, jnp.stack(upd, axis=0))


def _mlp2_pool(h, p0, p1, wa, ba, wb, bb, batch3):
    blk = lambda i: (i, 0)
    fixed = lambda i: (0, 0)
    return pl.pallas_call(
        _mlp2_pool_block,
        grid=(N_BLKS,),
        in_specs=[
            pl.BlockSpec((ROWS_BLK, D), blk),
            pl.BlockSpec((ROWS_BLK, D), blk),
            pl.BlockSpec((ROWS_BLK, D), blk),
            pl.BlockSpec((D, D), fixed),
            pl.BlockSpec((1, D), fixed),
            pl.BlockSpec((D, D), fixed),
            pl.BlockSpec((1, D), fixed),
            pl.BlockSpec((ROWS_BLK, 1), lambda i: (i, 0)),
        ],
        out_specs=pl.BlockSpec((G, D), fixed),
        out_shape=jax.ShapeDtypeStruct((G, D), jnp.float32),
    )(h, p0, p1, wa, ba, wb, bb, batch3)


def _final_block(pool_ref, wf1_ref, bf1_ref, wf2_ref, bf2_ref, out_ref):
    pooled = pool_ref[...]
    pooled = jnp.where(jnp.isfinite(pooled), pooled, 0.0)
    z = jnp.maximum(
        jnp.dot(pooled, wf1_ref[...], preferred_element_type=jnp.float32)
        + bf1_ref[...], 0.0)
    o = (jnp.dot(z, wf2_ref[...], preferred_element_type=jnp.float32)
         + bf2_ref[...])
    m = jnp.max(o, axis=1, keepdims=True)
    s = o - m
    lse = jnp.log(jnp.sum(jnp.exp(s), axis=1, keepdims=True))
    out_ref[...] = s - lse


def _final(pooled, wf1, bf1, wf2, bf2):
    return pl.pallas_call(
        _final_block,
        out_shape=jax.ShapeDtypeStruct((G, NCLS), jnp.float32),
    )(pooled, wf1, bf1, wf2, bf2)


def kernel(x, edge_index, batch, W1a, b1a, W1b, b1b, W2a, b2a, W2b, b2b,
           Wf1, bf1, Wf2, bf2):
    src = edge_index[0]
    dst = edge_index[1]
    s2 = jnp.concatenate([src, dst])
    d2 = jnp.concatenate([dst, src])
    key = lax.sort(d2 * N + s2, is_stable=False)
    dup = jnp.concatenate([jnp.zeros((1,), bool), key[1:] == key[:-1]])
    gs = (key % N).astype(jnp.int32)
    sd = jnp.where(dup, N, key // N).astype(jnp.int32)

    e2 = gs.shape[0]
    tpw = -(-e2 // (NW * CH))
    tpw = tpw + (tpw % 2)
    e2p = NW * tpw * CH
    pad = e2p - e2
    gs = jnp.concatenate([gs, jnp.zeros((pad,), jnp.int32)])
    sd = jnp.concatenate([sd, jnp.full((pad,), N, jnp.int32)])

    agg = _make_agg(tpw)

    ba1 = b1a.reshape(1, D)
    bb1 = b1b.reshape(1, D)
    ba2 = b2a.reshape(1, D)
    bb2 = b2b.reshape(1, D)
    batch3 = batch.astype(jnp.int32).reshape(N, 1)

    p0, p1 = agg(x, gs, sd)
    h = _mlp1(x, p0, p1, W1a, ba1, W1b, bb1)
    q0, q1 = agg(h, gs, sd)
    pooled = _mlp2_pool(h, q0, q1, W2a, ba2, W2b, bb2, batch3)
    out = _final(pooled, Wf1, bf1.reshape(1, 64), Wf2, bf2.reshape(1, NCLS))
    return out
